# baseline (speedup 1.0000x reference)
import functools

import jax
import jax.numpy as jnp
from jax import lax
from jax.experimental import pallas as pl
from jax.experimental.pallas import tpu as pltpu
from jax.experimental.pallas import tpu_sc as plsc

NC = 2
NS = 16
NW = NC * NS
L = 16


@functools.lru_cache(maxsize=None)
def _make_kernel(B, S, V, D, C):
    s_per_w = S // NW
    chunks = s_per_w // C
    ncol = D // L

    mesh = plsc.VectorSubcoreMesh(core_axis_name="c", subcore_axis_name="s")

    @functools.partial(
        pl.kernel,
        mesh=mesh,
        out_type=jax.ShapeDtypeStruct((B * S, D), jnp.float32),
        scratch_types=[
            pltpu.VMEM((B, s_per_w), jnp.int32),
            pltpu.VMEM((2 * B, C, D), jnp.float32),
            pltpu.VMEM((4, C, D), jnp.float32),
            pltpu.SemaphoreType.DMA((2 * B,)),
            pltpu.SemaphoreType.DMA((2 * B,)),
            pltpu.SemaphoreType.DMA((4,)),
        ],
    )
    def emb_kernel(ids_hbm, tok_hbm, pos_hbm, out_hbm, idx_v, tokb, posb, gsem, ssem, psem):
        wid = lax.axis_index("s") * NC + lax.axis_index("c")
        s0 = wid * s_per_w

        for b in range(B):
            pltpu.async_copy(ids_hbm.at[pl.ds(b * S + s0, s_per_w)], idx_v.at[b], ssem.at[b])
        for b in range(B):
            pltpu.make_async_copy(
                ids_hbm.at[pl.ds(b * S + s0, s_per_w)], idx_v.at[b], ssem.at[b]
            ).wait()

        def gather(k, b, u):
            pltpu.async_copy(
                tok_hbm.at[idx_v.at[b, pl.ds(k * C, C)]], tokb.at[u], gsem.at[u]
            )

        def gather_wait(u):
            pltpu.make_async_copy(
                tok_hbm.at[pl.ds(0, C)], tokb.at[u], gsem.at[u]
            ).wait()

        def scatter(k, b, u):
            pltpu.async_copy(
                tokb.at[u], out_hbm.at[pl.ds(b * S + s0 + k * C, C)], ssem.at[u]
            )

        def scatter_wait(u):
            pltpu.make_async_copy(
                tokb.at[u], out_hbm.at[pl.ds(0, C)], ssem.at[u]
            ).wait()

        def pos_load(k, pb):
            pltpu.async_copy(pos_hbm.at[pl.ds(s0 + k * C, C)], posb.at[pb], psem.at[pb])

        def pos_wait(pb):
            pltpu.make_async_copy(
                pos_hbm.at[pl.ds(s0, C)], posb.at[pb], psem.at[pb]
            ).wait()

        pos_load(0, 0)
        pos_load(1, 1)
        for b in range(B):
            gather(0, b, b)
        gather(1, 0, 4)
        gather(1, 1, 5)

        def outer(j, carry):
            for ss in range(0, 4 * B, 2):
                k = 4 * j + ss // B
                b = ss % B
                pb = ss // B
                us = ss % (2 * B)

                if b == 0:
                    pos_wait(pb)
                    if ss < 2 * B:
                        pos_load(k + 2, (pb + 2) % 4)
                    else:
                        @pl.when(j < chunks // 4 - 1)
                        def _():
                            pos_load(k + 2, (pb + 2) % 4)

                gather_wait(us)
                gather_wait(us + 1)

                kr = 4 * j + (ss + 6) // 4
                br = (ss + 6) % 4
                for p in range(2):
                    u2 = (us + 6 + p) % (2 * B)
                    if ss == 0:
                        @pl.when(j >= 1)
                        def _():
                            scatter_wait(u2)
                        gather(kr, br + p, u2)
                    elif ss <= 2 * B:
                        scatter_wait(u2)
                        gather(kr, br + p, u2)
                    else:
                        @pl.when(j < chunks // 4 - 1)
                        def _():
                            scatter_wait(u2)
                            gather(kr, br + p, u2)

                def row_body(r, c2):
                    for c in range(ncol):
                        sl = pl.ds(c * L, L)
                        pv = posb[pb, r, sl]
                        for p in range(2):
                            tokb[us + p, r, sl] = tokb[us + p, r, sl] + pv
                    return c2

                lax.fori_loop(0, C, row_body, 0)
                scatter(k, b, us)
                scatter(k, b + 1, us + 1)
            return carry

        lax.fori_loop(0, chunks // 4, outer, 0)

        for u in range(2 * B):
            scatter_wait(u)

    return emb_kernel


def kernel(input_ids, token_embeddings, position_embeddings):
    B, S = input_ids.shape
    V, D = token_embeddings.shape
    ids = input_ids.reshape(-1).astype(jnp.int32)
    k = _make_kernel(B, S, V, D, 8)
    out = k(ids, token_embeddings, position_embeddings)
    return out.reshape(B, S, D)
